# Initial kernel scaffold; baseline (speedup 1.0000x reference)
import jax
import jax.numpy as jnp
from jax.experimental import pallas as pl


def kernel(x, table, lin_table, W1, b1, W2, b2, W3, b3):
    raise NotImplementedError("write your pallas kernel here")


# speedup vs baseline: 2.0137x; 2.0137x over previous
import functools

import numpy as np

import jax
import jax.numpy as jnp
from jax import lax
from jax.experimental import pallas as pl
from jax.experimental.pallas import tpu as pltpu
from jax.experimental.pallas import tpu_sc as plsc

_B = 16384
_F = 26
_D = 32
_FD = _F * _D
_H1, _H2 = 256, 128

_NW = 32
_RPT = _B * _F // _NW
_CW = 128
_NCH = _RPT // _CW
_GRP = 8
_NG = _NCH // _GRP

_mesh = plsc.VectorSubcoreMesh(core_axis_name="c", subcore_axis_name="s")


def _sc_gather_body(xe_hbm, xl_hbm, table_hbm, lin_hbm,
                    emb_out, lin_out,
                    idx_e, idx_l, ebuf, linv, sem_e, sem_l):
    c = lax.axis_index("c")
    s = lax.axis_index("s")
    wid = s * 2 + c
    base = wid * _RPT

    pltpu.sync_copy(xe_hbm.at[wid], idx_e)
    pltpu.sync_copy(xl_hbm.at[wid], idx_l)

    def emb_group(g, carry):
        cps = []
        for j in range(_GRP):
            cps.append(pltpu.async_copy(
                table_hbm.at[idx_e.at[g * _GRP + j]],
                ebuf.at[pl.ds(j * _CW, _CW)],
                sem_e))
        for cp in cps:
            cp.wait()
        pltpu.sync_copy(
            ebuf, emb_out.at[pl.ds(base + g * (_GRP * _CW), _GRP * _CW)])
        return carry

    lax.fori_loop(0, _NG, emb_group, 0)

    def lin_group(g, carry):
        cps = []
        for j in range(_GRP):
            k = g * _GRP + j
            cps.append(pltpu.async_copy(
                lin_hbm.at[idx_l.at[k]],
                linv.at[pl.ds(k * _CW, _CW)],
                sem_l))
        for cp in cps:
            cp.wait()
        return carry

    lax.fori_loop(0, _NG, lin_group, 0)
    pltpu.sync_copy(linv, lin_out.at[pl.ds(base, _RPT)])


_sc_gather = functools.partial(
    pl.kernel,
    mesh=_mesh,
    compiler_params=pltpu.CompilerParams(use_tc_tiling_on_sc=False),
    out_type=[
        jax.ShapeDtypeStruct((_B * _F, _D), jnp.float32),
        jax.ShapeDtypeStruct((_B * _F,), jnp.float32),
    ],
    scratch_types=[
        pltpu.VMEM((_NCH, _CW), jnp.int32),
        pltpu.VMEM((_NCH, _CW), jnp.int32),
        pltpu.VMEM((_GRP * _CW, _D), jnp.float32),
        pltpu.VMEM((_RPT,), jnp.float32),
        pltpu.SemaphoreType.DMA,
        pltpu.SemaphoreType.DMA,
    ],
)(_sc_gather_body)


_BB = 512


def _tc_body(emb_ref, lin_ref, w1_ref, b1_ref, w2_ref, b2_ref,
             w3_ref, b3_ref, s_ref, out_ref):
    e = emb_ref[...]
    h1 = jnp.maximum(
        jnp.dot(e, w1_ref[...], preferred_element_type=jnp.float32)
        + b1_ref[...], 0.0)
    h2 = jnp.maximum(
        jnp.dot(h1, w2_ref[...], preferred_element_type=jnp.float32)
        + b2_ref[...], 0.0)
    dnn = jnp.sum(h2 * w3_ref[...], axis=1, keepdims=True)
    fs = jnp.dot(e, s_ref[...], preferred_element_type=jnp.float32)
    fm = 0.5 * (jnp.sum(fs * fs, axis=1, keepdims=True)
                - jnp.sum(e * e, axis=1, keepdims=True))
    linsum = jnp.sum(lin_ref[...], axis=0)[:, None]
    z = linsum + fm + dnn + b3_ref[...]
    out_ref[...] = 1.0 / (1.0 + jnp.exp(-z))


_tc_call = pl.pallas_call(
    _tc_body,
    grid=(_B // _BB,),
    in_specs=[
        pl.BlockSpec((_BB, _FD), lambda i: (i, 0)),
        pl.BlockSpec((_F, _BB), lambda i: (0, i)),
        pl.BlockSpec((_FD, _H1), lambda i: (0, 0)),
        pl.BlockSpec((1, _H1), lambda i: (0, 0)),
        pl.BlockSpec((_H1, _H2), lambda i: (0, 0)),
        pl.BlockSpec((1, _H2), lambda i: (0, 0)),
        pl.BlockSpec((1, _H2), lambda i: (0, 0)),
        pl.BlockSpec((1, 1), lambda i: (0, 0)),
        pl.BlockSpec((_FD, _H2), lambda i: (0, 0)),
    ],
    out_specs=pl.BlockSpec((_BB, 1), lambda i: (i, 0)),
    out_shape=jax.ShapeDtypeStruct((_B, 1), jnp.float32),
)

_S = np.concatenate(
    [np.tile(np.eye(_D, dtype=np.float32), (_F, 1)),
     np.zeros((_FD, _H2 - _D), np.float32)], axis=1)


def kernel(x, table, lin_table, W1, b1, W2, b2, W3, b3):
    xe = x.reshape(_NW, _NCH, _CW)
    xl = x.T.reshape(_NW, _NCH, _CW)
    emb_flat, lin_flat = _sc_gather(xe, xl, table, lin_table.reshape(-1))
    emb = emb_flat.reshape(_B, _FD)
    ling = lin_flat.reshape(_F, _B)
    return _tc_call(emb, ling, W1, b1.reshape(1, -1), W2, b2.reshape(1, -1),
                    W3.reshape(1, -1), b3.reshape(1, 1), jnp.asarray(_S))


# speedup vs baseline: 2.0515x; 1.0188x over previous
import functools

import numpy as np

import jax
import jax.numpy as jnp
from jax import lax
from jax.experimental import pallas as pl
from jax.experimental.pallas import tpu as pltpu
from jax.experimental.pallas import tpu_sc as plsc

_B = 16384
_F = 26
_D = 32
_FD = _F * _D
_H1, _H2 = 256, 128

_NW = 32
_RPT = _B * _F // _NW
_CW = 128
_NCH = _RPT // _CW
_GRP = 8
_NG = _NCH // _GRP

_mesh = plsc.VectorSubcoreMesh(core_axis_name="c", subcore_axis_name="s")


def _sc_gather_body(xe_hbm, xl_hbm, table_hbm, lin_hbm,
                    emb_out, lin_out,
                    idx_e, idx_l, eb0, eb1, linv, sem_e, sem_l, sem_w):
    c = lax.axis_index("c")
    s = lax.axis_index("s")
    wid = s * 2 + c
    base = wid * _NCH

    pltpu.sync_copy(xe_hbm.at[wid], idx_e)
    pltpu.sync_copy(xl_hbm.at[wid], idx_l)

    lin_cps = [
        pltpu.async_copy(lin_hbm.at[idx_l.at[k]], linv.at[k], sem_l)
        for k in range(_NCH)
    ]

    ebufs = (eb0, eb1)

    def fire_gather(g):
        return [
            pltpu.async_copy(
                table_hbm.at[idx_e.at[g * _GRP + j]],
                ebufs[g % 2].at[j], sem_e)
            for j in range(_GRP)
        ]

    def fire_write(g):
        return pltpu.async_copy(
            ebufs[g % 2],
            emb_out.at[pl.ds(base + g * _GRP, _GRP)], sem_w)

    g_cp = {0: fire_gather(0)}
    w_cp = [None, None]
    for g in range(_NG):
        for cp in g_cp.pop(g):
            cp.wait()
        w_cp[g % 2] = fire_write(g)
        if g + 1 < _NG:
            if w_cp[(g + 1) % 2] is not None:
                w_cp[(g + 1) % 2].wait()
            g_cp[g + 1] = fire_gather(g + 1)
    for cp in lin_cps:
        cp.wait()
    pltpu.sync_copy(linv, lin_out.at[pl.ds(base, _NCH)])
    for cp in w_cp:
        if cp is not None:
            cp.wait()


_sc_gather = functools.partial(
    pl.kernel,
    mesh=_mesh,
    compiler_params=pltpu.CompilerParams(use_tc_tiling_on_sc=False),
    out_type=[
        jax.ShapeDtypeStruct((_B * _F // _CW, _CW, _D), jnp.float32),
        jax.ShapeDtypeStruct((_B * _F // _CW, _CW), jnp.float32),
    ],
    scratch_types=[
        pltpu.VMEM((_NCH, _CW), jnp.int32),
        pltpu.VMEM((_NCH, _CW), jnp.int32),
        pltpu.VMEM((_GRP, _CW, _D), jnp.float32),
        pltpu.VMEM((_GRP, _CW, _D), jnp.float32),
        pltpu.VMEM((_NCH, _CW), jnp.float32),
        pltpu.SemaphoreType.DMA,
        pltpu.SemaphoreType.DMA,
        pltpu.SemaphoreType.DMA,
    ],
)(_sc_gather_body)


_BB = 512


def _tc_body(emb_ref, lin_ref, w1_ref, b1_ref, w2_ref, b2_ref,
             w3_ref, b3_ref, s_ref, out_ref):
    e = emb_ref[...]
    h1 = jnp.maximum(
        jnp.dot(e, w1_ref[...], preferred_element_type=jnp.float32)
        + b1_ref[...], 0.0)
    h2 = jnp.maximum(
        jnp.dot(h1, w2_ref[...], preferred_element_type=jnp.float32)
        + b2_ref[...], 0.0)
    dnn = jnp.sum(h2 * w3_ref[...], axis=1, keepdims=True)
    fs = jnp.dot(e, s_ref[...], preferred_element_type=jnp.float32)
    fm = 0.5 * (jnp.sum(fs * fs, axis=1, keepdims=True)
                - jnp.sum(e * e, axis=1, keepdims=True))
    linsum = jnp.sum(lin_ref[...], axis=0)[:, None]
    z = linsum + fm + dnn + b3_ref[...]
    out_ref[...] = 1.0 / (1.0 + jnp.exp(-z))


_tc_call = pl.pallas_call(
    _tc_body,
    grid=(_B // _BB,),
    in_specs=[
        pl.BlockSpec((_BB, _FD), lambda i: (i, 0)),
        pl.BlockSpec((_F, _BB), lambda i: (0, i)),
        pl.BlockSpec((_FD, _H1), lambda i: (0, 0)),
        pl.BlockSpec((1, _H1), lambda i: (0, 0)),
        pl.BlockSpec((_H1, _H2), lambda i: (0, 0)),
        pl.BlockSpec((1, _H2), lambda i: (0, 0)),
        pl.BlockSpec((1, _H2), lambda i: (0, 0)),
        pl.BlockSpec((1, 1), lambda i: (0, 0)),
        pl.BlockSpec((_FD, _H2), lambda i: (0, 0)),
    ],
    out_specs=pl.BlockSpec((_BB, 1), lambda i: (i, 0)),
    out_shape=jax.ShapeDtypeStruct((_B, 1), jnp.float32),
)

_S = np.concatenate(
    [np.tile(np.eye(_D, dtype=np.float32), (_F, 1)),
     np.zeros((_FD, _H2 - _D), np.float32)], axis=1)


def kernel(x, table, lin_table, W1, b1, W2, b2, W3, b3):
    xe = x.reshape(_NW, _NCH, _CW)
    xl = x.T.reshape(_NW, _NCH, _CW)
    emb_flat, lin_flat = _sc_gather(xe, xl, table, lin_table.reshape(-1))
    emb = emb_flat.reshape(_B, _FD)
    ling = lin_flat.reshape(_F, _B)
    return _tc_call(emb, ling, W1, b1.reshape(1, -1), W2, b2.reshape(1, -1),
                    W3.reshape(1, -1), b3.reshape(1, 1), jnp.asarray(_S))
